# Initial kernel scaffold; baseline (speedup 1.0000x reference)
import functools

import jax
import jax.numpy as jnp
from jax import lax
from jax.experimental import pallas as pl
from jax.experimental.pallas import tpu as pltpu
from jax.experimental.pallas import tpu_sc as plsc

ND = 128
ED = 16
OD = 128
N_NODES = 10000
N_EDGES = 320000

NC = 2
NS = 16
NW = NC * NS
EPW = N_EDGES // NW
CH = 80
NCHUNK = EPW // CH
RPS = N_NODES // NS
ZROWS = 125

_SC_MESH = plsc.VectorSubcoreMesh(core_axis_name="c", subcore_axis_name="s")


def _proj_body(x_ref, ws_ref, wd_ref, ps_ref, pd_ref):
    xb = x_ref[...]
    ps_ref[...] = jnp.dot(xb, ws_ref[...], preferred_element_type=jnp.float32)
    pd_ref[...] = jnp.dot(xb, wd_ref[...], preferred_element_type=jnp.float32)


def _node_proj(x, w1s, w1d):
    bn = 2000
    return pl.pallas_call(
        _proj_body,
        grid=(N_NODES // bn,),
        in_specs=[
            pl.BlockSpec((bn, ND), lambda i: (i, 0)),
            pl.BlockSpec((ND, OD), lambda i: (0, 0)),
            pl.BlockSpec((ND, OD), lambda i: (0, 0)),
        ],
        out_specs=[
            pl.BlockSpec((bn, OD), lambda i: (i, 0)),
            pl.BlockSpec((bn, OD), lambda i: (i, 0)),
        ],
        out_shape=[
            jax.ShapeDtypeStruct((N_NODES, OD), jnp.float32),
            jax.ShapeDtypeStruct((N_NODES, OD), jnp.float32),
        ],
    )(x, w1s, w1d)


@functools.partial(
    pl.kernel,
    out_type=jax.ShapeDtypeStruct((N_EDGES, OD), jnp.float32),
    mesh=_SC_MESH,
    scratch_types=[
        pltpu.VMEM((CH,), jnp.int32),
        pltpu.VMEM((CH,), jnp.int32),
        pltpu.VMEM((CH, OD), jnp.float32),
        pltpu.VMEM((CH, OD), jnp.float32),
        pltpu.SemaphoreType.DMA,
    ],
)
def _gather_sum(s_hbm, d_hbm, ps_hbm, pd_hbm, g_hbm, sidx, didx, bufa, bufb, sem):
    wid = lax.axis_index("s") * NC + lax.axis_index("c")
    base = wid * EPW

    def chunk(ci, carry):
        off = base + ci * CH
        pltpu.sync_copy(s_hbm.at[pl.ds(off, CH)], sidx)
        pltpu.sync_copy(d_hbm.at[pl.ds(off, CH)], didx)
        ca = pltpu.async_copy(ps_hbm.at[sidx], bufa, sem)
        cb = pltpu.async_copy(pd_hbm.at[didx], bufb, sem)
        ca.wait()
        cb.wait()

        def row(i, c2):
            def col(j, c3):
                sl = pl.ds(j * 16, 16)
                bufa[i, sl] = bufa[i, sl] + bufb[i, sl]
                return c3
            return lax.fori_loop(0, OD // 16, col, c2, unroll=True)

        lax.fori_loop(0, CH, row, carry)
        pltpu.sync_copy(bufa, g_hbm.at[pl.ds(off, CH)])
        return carry

    lax.fori_loop(0, NCHUNK, chunk, 0)


def _msg_body(g_ref, ea_ref, w1e_ref, b1_ref, w2_ref, b2_ref, m_ref):
    pe = jnp.dot(ea_ref[...], w1e_ref[...], preferred_element_type=jnp.float32)
    h1 = jnp.maximum(g_ref[...] + pe + b1_ref[...], 0.0)
    m_ref[...] = (
        jnp.dot(h1, w2_ref[...], preferred_element_type=jnp.float32) + b2_ref[...]
    )


def _edge_mlp(g, ea, w1e, b1, w2, b2):
    be = 3200
    return pl.pallas_call(
        _msg_body,
        grid=(N_EDGES // be,),
        in_specs=[
            pl.BlockSpec((be, OD), lambda i: (i, 0)),
            pl.BlockSpec((be, ED), lambda i: (i, 0)),
            pl.BlockSpec((ED, OD), lambda i: (0, 0)),
            pl.BlockSpec((1, OD), lambda i: (0, 0)),
            pl.BlockSpec((OD, OD), lambda i: (0, 0)),
            pl.BlockSpec((1, OD), lambda i: (0, 0)),
        ],
        out_specs=pl.BlockSpec((be, OD), lambda i: (i, 0)),
        out_shape=jax.ShapeDtypeStruct((N_EDGES, OD), jnp.float32),
    )(g, ea, w1e, b1, w2, b2)


@functools.partial(
    pl.kernel,
    out_type=jax.ShapeDtypeStruct((NC, N_NODES, OD), jnp.float32),
    mesh=_SC_MESH,
    scratch_types=[
        pltpu.VMEM((CH,), jnp.int32),
        pltpu.VMEM((CH, OD), jnp.float32),
        pltpu.VMEM((ZROWS, OD), jnp.float32),
        pltpu.VMEM_SHARED((N_NODES, OD), jnp.float32),
        pltpu.SemaphoreType.DMA,
    ],
)
def _scatter_add(d_hbm, m_hbm, out_hbm, idx, mbuf, zbuf, acc, sem):
    del sem
    cid = lax.axis_index("c")
    sid = lax.axis_index("s")
    wid = sid * NC + cid

    def zrow(i, c):
        def zcol(j, c2):
            zbuf[i, pl.ds(j * 16, 16)] = jnp.zeros((16,), jnp.float32)
            return c2
        return lax.fori_loop(0, OD // 16, zcol, c, unroll=True)

    lax.fori_loop(0, ZROWS, zrow, 0)
    for k in range(RPS // ZROWS):
        pltpu.sync_copy(zbuf, acc.at[pl.ds(sid * RPS + k * ZROWS, ZROWS)])
    plsc.subcore_barrier()

    base = wid * EPW

    def chunk(ci, c):
        off = base + ci * CH
        pltpu.sync_copy(d_hbm.at[pl.ds(off, CH)], idx)
        pltpu.sync_copy(m_hbm.at[pl.ds(off, CH)], mbuf)
        pltpu.sync_copy(mbuf, acc.at[idx], add=True)
        return c

    lax.fori_loop(0, NCHUNK, chunk, 0)
    plsc.subcore_barrier()

    for k in range(RPS // ZROWS):
        sl = pl.ds(sid * RPS + k * ZROWS, ZROWS)
        pltpu.sync_copy(acc.at[sl], zbuf)
        pltpu.sync_copy(zbuf, out_hbm.at[cid, sl])


def _gru_body(a0_ref, a1_ref, x_ref, wih_ref, whh_ref, bih_ref, bhh_ref, o_ref):
    agg = a0_ref[0] + a1_ref[0]
    xb = x_ref[...]
    gi = jnp.dot(agg, wih_ref[...], preferred_element_type=jnp.float32) + bih_ref[...]
    gh = jnp.dot(xb, whh_ref[...], preferred_element_type=jnp.float32) + bhh_ref[...]
    r = jax.nn.sigmoid(gi[:, :ND] + gh[:, :ND])
    z = jax.nn.sigmoid(gi[:, ND:2 * ND] + gh[:, ND:2 * ND])
    n = jnp.tanh(gi[:, 2 * ND:] + r * gh[:, 2 * ND:])
    o_ref[...] = (1.0 - z) * n + z * xb


def _gru_update(acc, x, wih, whh, bih, bhh):
    bn = 2000
    return pl.pallas_call(
        _gru_body,
        grid=(N_NODES // bn,),
        in_specs=[
            pl.BlockSpec((1, bn, ND), lambda i: (0, i, 0)),
            pl.BlockSpec((1, bn, ND), lambda i: (1, i, 0)),
            pl.BlockSpec((bn, ND), lambda i: (i, 0)),
            pl.BlockSpec((ND, 3 * ND), lambda i: (0, 0)),
            pl.BlockSpec((ND, 3 * ND), lambda i: (0, 0)),
            pl.BlockSpec((1, 3 * ND), lambda i: (0, 0)),
            pl.BlockSpec((1, 3 * ND), lambda i: (0, 0)),
        ],
        out_specs=pl.BlockSpec((bn, ND), lambda i: (i, 0)),
        out_shape=jax.ShapeDtypeStruct((N_NODES, ND), jnp.float32),
    )(acc, acc, x, wih, whh, bih, bhh)


def kernel(x, ei, ea, W1, b1, W2, b2, Wih, Whh, bih, bhh):
    s = ei[0].astype(jnp.int32)
    d = ei[1].astype(jnp.int32)
    w1s = W1[:ND]
    w1d = W1[ND:2 * ND]
    w1e = W1[2 * ND:]

    ps, pd = _node_proj(x, w1s, w1d)
    g = _gather_sum(s, d, ps, pd)
    m = _edge_mlp(g, ea, w1e, b1.reshape(1, OD), W2, b2.reshape(1, OD))
    acc = _scatter_add(d, m)
    return _gru_update(acc, x, Wih, Whh, bih.reshape(1, 3 * ND),
                       bhh.reshape(1, 3 * ND))


# speedup vs baseline: 2.8083x; 2.8083x over previous
import functools

import jax
import jax.numpy as jnp
from jax import lax
from jax.experimental import pallas as pl
from jax.experimental.pallas import tpu as pltpu
from jax.experimental.pallas import tpu_sc as plsc

ND = 128
ED = 16
OD = 128
N_NODES = 10000
N_EDGES = 320000

NC = 2
NS = 16
NW = NC * NS
EPW = N_EDGES // NW
CH = 80
NCHUNK = EPW // CH
NACC = N_NODES // CH

@functools.cache
def _sc_mesh():
    return plsc.VectorSubcoreMesh(
        core_axis_name="c", subcore_axis_name="s", num_cores=NC, num_subcores=NS
    )


def _proj_body(x_ref, ws_ref, wd_ref, ps_ref, pd_ref):
    xb = x_ref[...]
    ps_ref[...] = jnp.dot(xb, ws_ref[...], preferred_element_type=jnp.float32)
    pd_ref[...] = jnp.dot(xb, wd_ref[...], preferred_element_type=jnp.float32)


def _node_proj(x, w1s, w1d):
    bn = 2000
    return pl.pallas_call(
        _proj_body,
        grid=(N_NODES // bn,),
        in_specs=[
            pl.BlockSpec((bn, ND), lambda i: (i, 0)),
            pl.BlockSpec((ND, OD), lambda i: (0, 0)),
            pl.BlockSpec((ND, OD), lambda i: (0, 0)),
        ],
        out_specs=[
            pl.BlockSpec((bn, OD), lambda i: (i, 0)),
            pl.BlockSpec((bn, OD), lambda i: (i, 0)),
        ],
        out_shape=[
            jax.ShapeDtypeStruct((N_NODES, OD), jnp.float32),
            jax.ShapeDtypeStruct((N_NODES, OD), jnp.float32),
        ],
    )(x, w1s, w1d)


@functools.cache
def _build_gather_sum():
    return pl.kernel(
        _gather_sum_body,
        out_type=jax.ShapeDtypeStruct((N_EDGES, OD), jnp.float32),
        mesh=_sc_mesh(),
        scratch_types=[
            pltpu.VMEM((CH,), jnp.int32),
            pltpu.VMEM((CH,), jnp.int32),
            pltpu.VMEM((CH, OD), jnp.float32),
            pltpu.VMEM((CH, OD), jnp.float32),
            pltpu.SemaphoreType.DMA,
        ],
    )


def _gather_sum_body(s_hbm, d_hbm, ps_hbm, pd_hbm, g_hbm, sidx, didx, bufa, bufb, sem):
    wid = lax.axis_index("s") * NC + lax.axis_index("c")
    base = wid * EPW

    def chunk(ci, carry):
        off = base + ci * CH
        pltpu.sync_copy(s_hbm.at[pl.ds(off, CH)], sidx)
        pltpu.sync_copy(d_hbm.at[pl.ds(off, CH)], didx)
        ca = pltpu.async_copy(ps_hbm.at[sidx], bufa, sem)
        cb = pltpu.async_copy(pd_hbm.at[didx], bufb, sem)
        ca.wait()
        cb.wait()

        def row(i, c2):
            def col(j, c3):
                sl = pl.ds(j * 16, 16)
                bufa[i, sl] = bufa[i, sl] + bufb[i, sl]
                return c3
            return lax.fori_loop(0, OD // 16, col, c2, unroll=True)

        lax.fori_loop(0, CH, row, carry)
        pltpu.sync_copy(bufa, g_hbm.at[pl.ds(off, CH)])
        return carry

    lax.fori_loop(0, NCHUNK, chunk, 0)


def _msg_body(g_ref, ea_ref, w1e_ref, b1_ref, w2_ref, b2_ref, m_ref):
    pe = jnp.dot(ea_ref[...], w1e_ref[...], preferred_element_type=jnp.float32)
    h1 = jnp.maximum(g_ref[...] + pe + b1_ref[...], 0.0)
    m_ref[...] = (
        jnp.dot(h1, w2_ref[...], preferred_element_type=jnp.float32) + b2_ref[...]
    )


def _edge_mlp(g, ea, w1e, b1, w2, b2):
    be = 3200
    return pl.pallas_call(
        _msg_body,
        grid=(N_EDGES // be,),
        in_specs=[
            pl.BlockSpec((be, OD), lambda i: (i, 0)),
            pl.BlockSpec((be, ED), lambda i: (i, 0)),
            pl.BlockSpec((ED, OD), lambda i: (0, 0)),
            pl.BlockSpec((1, OD), lambda i: (0, 0)),
            pl.BlockSpec((OD, OD), lambda i: (0, 0)),
            pl.BlockSpec((1, OD), lambda i: (0, 0)),
        ],
        out_specs=pl.BlockSpec((be, OD), lambda i: (i, 0)),
        out_shape=jax.ShapeDtypeStruct((N_EDGES, OD), jnp.float32),
    )(g, ea, w1e, b1, w2, b2)


@functools.cache
def _build_scatter_add():
    return pl.kernel(
        _scatter_add_body,
        out_type=jax.ShapeDtypeStruct((NC, N_NODES, OD), jnp.float32),
        mesh=_sc_mesh(),
        scratch_types=[
            pltpu.VMEM((CH,), jnp.int32),
            pltpu.VMEM((CH, OD), jnp.float32),
            pltpu.VMEM((CH, OD), jnp.float32),
            pltpu.VMEM_SHARED((N_NODES, OD), jnp.float32),
            pltpu.SemaphoreType.DMA,
        ],
    )


def _scatter_add_body(d_hbm, m_hbm, out_hbm, idx, mbuf, zbuf, acc, sem):
    del sem
    cid = lax.axis_index("c")
    sid = lax.axis_index("s")
    wid = sid * NC + cid

    def zrow(i, c):
        def zcol(j, c2):
            zbuf[i, pl.ds(j * 16, 16)] = jnp.zeros((16,), jnp.float32)
            return c2
        return lax.fori_loop(0, OD // 16, zcol, c, unroll=True)

    lax.fori_loop(0, CH, zrow, 0)
    for j in range((NACC + NS - 1) // NS):
        c = sid + NS * j
        @pl.when(c < NACC)
        def _():
            pltpu.sync_copy(zbuf, acc.at[pl.ds(c * CH, CH)])
    plsc.subcore_barrier()

    base = wid * EPW

    def chunk(ci, c):
        off = base + ci * CH
        pltpu.sync_copy(d_hbm.at[pl.ds(off, CH)], idx)
        pltpu.sync_copy(m_hbm.at[pl.ds(off, CH)], mbuf)
        pltpu.sync_copy(mbuf, acc.at[idx], add=True)
        return c

    lax.fori_loop(0, NCHUNK, chunk, 0)
    plsc.subcore_barrier()

    for j in range((NACC + NS - 1) // NS):
        c = sid + NS * j
        @pl.when(c < NACC)
        def _():
            sl = pl.ds(c * CH, CH)
            pltpu.sync_copy(acc.at[sl], zbuf)
            pltpu.sync_copy(zbuf, out_hbm.at[cid, sl])


def _gru_body(a0_ref, a1_ref, x_ref, wih_ref, whh_ref, bih_ref, bhh_ref, o_ref):
    agg = a0_ref[0] + a1_ref[0]
    xb = x_ref[...]
    gi = jnp.dot(agg, wih_ref[...], preferred_element_type=jnp.float32) + bih_ref[...]
    gh = jnp.dot(xb, whh_ref[...], preferred_element_type=jnp.float32) + bhh_ref[...]
    r = jax.nn.sigmoid(gi[:, :ND] + gh[:, :ND])
    z = jax.nn.sigmoid(gi[:, ND:2 * ND] + gh[:, ND:2 * ND])
    n = jnp.tanh(gi[:, 2 * ND:] + r * gh[:, 2 * ND:])
    o_ref[...] = (1.0 - z) * n + z * xb


def _gru_update(acc, x, wih, whh, bih, bhh):
    bn = 2000
    return pl.pallas_call(
        _gru_body,
        grid=(N_NODES // bn,),
        in_specs=[
            pl.BlockSpec((1, bn, ND), lambda i: (0, i, 0)),
            pl.BlockSpec((1, bn, ND), lambda i: (1, i, 0)),
            pl.BlockSpec((bn, ND), lambda i: (i, 0)),
            pl.BlockSpec((ND, 3 * ND), lambda i: (0, 0)),
            pl.BlockSpec((ND, 3 * ND), lambda i: (0, 0)),
            pl.BlockSpec((1, 3 * ND), lambda i: (0, 0)),
            pl.BlockSpec((1, 3 * ND), lambda i: (0, 0)),
        ],
        out_specs=pl.BlockSpec((bn, ND), lambda i: (i, 0)),
        out_shape=jax.ShapeDtypeStruct((N_NODES, ND), jnp.float32),
    )(acc, acc, x, wih, whh, bih, bhh)


def kernel(x, ei, ea, W1, b1, W2, b2, Wih, Whh, bih, bhh):
    s = ei[0].astype(jnp.int32)
    d = ei[1].astype(jnp.int32)
    w1s = W1[:ND]
    w1d = W1[ND:2 * ND]
    w1e = W1[2 * ND:]

    ps, pd = _node_proj(x, w1s, w1d)
    g = _build_gather_sum()(s, d, ps, pd)
    m = _edge_mlp(g, ea, w1e, b1.reshape(1, OD), W2, b2.reshape(1, OD))
    acc = _build_scatter_add()(d, m)
    return _gru_update(acc, x, Wih, Whh, bih.reshape(1, 3 * ND),
                       bhh.reshape(1, 3 * ND))
